# baseline (speedup 1.0000x reference)
import functools

import jax
import jax.numpy as jnp
from jax import lax
from jax.experimental import pallas as pl
from jax.experimental.pallas import tpu as pltpu
from jax.experimental.pallas import tpu_sc as plsc

K = 1024
C = 64
BN = 512


def _vq_dist_block(xf_ref, cb_ref, idx_ref, loss_ref):
    x = xf_ref[...]
    cb = cb_ref[...]
    cross2 = jax.lax.dot_general(
        x, cb * -2.0, (((1,), (1,)), ((), ())),
        preferred_element_type=jnp.float32)
    x_sq = jnp.sum(x * x, axis=-1, keepdims=True)
    e_sq = jnp.sum(cb * cb, axis=-1)
    s = cross2 + e_sq[None, :]
    m = jnp.min(s, axis=-1, keepdims=True)
    idx = jnp.argmin(s, axis=-1)
    idx_ref[0, 0, :] = idx

    @pl.when(pl.program_id(0) == 0)
    def _init():
        loss_ref[...] = jnp.zeros((1, 1), jnp.float32)

    loss_ref[...] += jnp.sum(m + x_sq).reshape(1, 1)


def _make_sc_gather(B, HW):
    info = plsc.get_sparse_core_info()
    NW = info.num_cores * info.num_subcores
    CSLAB = C // (NW // B)
    L = info.num_lanes
    mesh = plsc.VectorSubcoreMesh(core_axis_name="c", subcore_axis_name="s")

    @functools.partial(
        pl.kernel, mesh=mesh,
        compiler_params=pltpu.CompilerParams(use_tc_tiling_on_sc=False, needs_layout_passes=False),
        out_type=jax.ShapeDtypeStruct((B, C, HW), jnp.float32),
        scratch_types=[
            pltpu.VMEM((HW,), jnp.int32),
            pltpu.VMEM((CSLAB * HW,), jnp.float32),
            pltpu.VMEM((CSLAB, HW), jnp.float32),
        ],
    )
    def gather(cbt_hbm, idx_hbm, out_hbm, idx_v, cbt_v, rows_v):
        wid = lax.axis_index("s") * info.num_cores + lax.axis_index("c")
        b = wid // (NW // B)
        c0 = (wid % (NW // B)) * CSLAB
        pltpu.sync_copy(idx_hbm.at[pl.ds(b * HW, HW)], idx_v)
        pltpu.sync_copy(cbt_hbm.at[pl.ds(c0 * HW, CSLAB * HW)], cbt_v)

        def body(p, _):
            def inner(j, _):
                iv = idx_v[pl.ds(j * L, L)] + p * HW
                g = plsc.load_gather(cbt_v, [iv])
                rows_v[p, pl.ds(j * L, L)] = g
                return 0
            return lax.fori_loop(0, HW // L, inner, 0, unroll=8)

        lax.fori_loop(0, CSLAB, body, 0)
        pltpu.sync_copy(rows_v, out_hbm.at[b, pl.ds(c0, CSLAB)])

    return gather


@jax.jit
def kernel(x, codebook):
    B, Cc, H, W = x.shape
    N = B * H * W
    HW = H * W
    xf = jnp.transpose(x, (0, 2, 3, 1)).reshape(N, Cc)
    grid = N // BN
    idx3, loss2 = pl.pallas_call(
        _vq_dist_block,
        grid=(grid,),
        in_specs=[
            pl.BlockSpec((BN, C), lambda i: (i, 0)),
            pl.BlockSpec((K, C), lambda i: (0, 0)),
        ],
        out_specs=[
            pl.BlockSpec((1, 1, BN), lambda i: (i, 0, 0)),
            pl.BlockSpec((1, 1), lambda i: (0, 0)),
        ],
        out_shape=[
            jax.ShapeDtypeStruct((grid, 1, BN), jnp.int32),
            jax.ShapeDtypeStruct((1, 1), jnp.float32),
        ],
    )(xf, codebook)
    idx_flat = idx3.reshape(N)
    cbt = codebook.T.reshape(-1)
    quant3 = _make_sc_gather(B, HW)(cbt, idx_flat)
    loss = (loss2[0, 0] / (N * Cc)).astype(jnp.float32)
    quant_out = quant3.reshape(B, Cc, H, W)
    idx_emb = idx_flat.reshape(B, HW)
    return (quant_out, loss, loss, idx_emb)


# speedup vs baseline: 1.2170x; 1.2170x over previous
import functools

import jax
import jax.numpy as jnp
from jax import lax
from jax.experimental import pallas as pl
from jax.experimental.pallas import tpu as pltpu
from jax.experimental.pallas import tpu_sc as plsc

K = 1024
C = 64
BN = 512


def _vq_dist_block(xf_ref, cb_ref, idx_ref, loss_ref):
    x = xf_ref[...]
    cb = cb_ref[...]
    cross2 = jax.lax.dot_general(
        x, cb * -2.0, (((1,), (1,)), ((), ())),
        preferred_element_type=jnp.float32)
    x_sq = jnp.sum(x * x, axis=-1, keepdims=True)
    e_sq = jnp.sum(cb * cb, axis=-1)
    s = cross2 + e_sq[None, :]
    m = jnp.min(s, axis=-1, keepdims=True)
    idx = jnp.argmin(s, axis=-1)
    idx_ref[0, 0, :] = idx
    loss_ref[0, 0, :] = jnp.broadcast_to(jnp.sum(m + x_sq), (128,))


def _make_sc_gather(B):
    info = plsc.get_sparse_core_info()
    NW = info.num_cores * info.num_subcores
    b_per_w = B // NW
    mesh = plsc.VectorSubcoreMesh(core_axis_name="c", subcore_axis_name="s")

    @functools.partial(
        pl.kernel, mesh=mesh,
        compiler_params=pltpu.CompilerParams(use_tc_tiling_on_sc=False),
        out_type=jax.ShapeDtypeStruct((B, C), jnp.float32),
        scratch_types=[
            pltpu.VMEM((b_per_w,), jnp.int32),
            pltpu.VMEM((b_per_w, C), jnp.float32),
            pltpu.SemaphoreType.DMA,
        ],
    )
    def gather(table_hbm, idx_hbm, out_hbm, idx_v, rows_v, sem):
        wid = lax.axis_index("s") * info.num_cores + lax.axis_index("c")
        base = wid * b_per_w
        pltpu.sync_copy(idx_hbm.at[pl.ds(base, b_per_w)], idx_v)
        pltpu.async_copy(table_hbm.at[idx_v], rows_v, sem).wait()
        pltpu.sync_copy(rows_v, out_hbm.at[pl.ds(base, b_per_w)])

    return gather


@jax.jit
def kernel(x, codebook):
    B, Cc, H, W = x.shape
    N = B * H * W
    xf = jnp.transpose(x, (0, 2, 3, 1)).reshape(N, Cc)
    grid = N // BN
    idx3, loss2 = pl.pallas_call(
        _vq_dist_block,
        grid=(grid,),
        in_specs=[
            pl.BlockSpec((BN, C), lambda i: (i, 0)),
            pl.BlockSpec((K, C), lambda i: (0, 0)),
        ],
        out_specs=[
            pl.BlockSpec((1, 1, BN), lambda i: (i, 0, 0)),
            pl.BlockSpec((1, 1, 128), lambda i: (i, 0, 0)),
        ],
        out_shape=[
            jax.ShapeDtypeStruct((grid, 1, BN), jnp.int32),
            jax.ShapeDtypeStruct((grid, 1, 128), jnp.float32),
        ],
        compiler_params=pltpu.CompilerParams(
            dimension_semantics=("parallel",)),
    )(xf, codebook)
    idx_flat = idx3.reshape(N)
    quant = _make_sc_gather(N)(codebook, idx_flat)
    loss = (jnp.sum(loss2[:, 0, 0]) / (N * Cc)).astype(jnp.float32)
    quant_out = jnp.transpose(quant.reshape(B, H, W, Cc), (0, 3, 1, 2))
    idx_emb = idx_flat.reshape(B, H * W)
    return (quant_out, loss, loss, idx_emb)


# speedup vs baseline: 1.5378x; 1.2636x over previous
import jax
import jax.numpy as jnp
from jax.experimental import pallas as pl
from jax.experimental.pallas import tpu as pltpu

K = 1024
C = 64
HW = 1024


def _vq_image_block(x_ref, cb_ref, qout_ref, idx_ref, loss_ref):
    xb = x_ref[0]
    cb = cb_ref[...]
    e_sq = jnp.sum(cb * cb, axis=-1)
    xb_aug = jnp.concatenate(
        [xb, jnp.ones((1, HW), jnp.float32)], axis=0)
    cb_aug = jnp.concatenate(
        [cb * -2.0, e_sq[:, None]], axis=1)
    s = jax.lax.dot_general(
        xb_aug, cb_aug, (((0,), (1,)), ((), ())),
        preferred_element_type=jnp.float32)
    m = jnp.min(s, axis=-1)
    idx = jnp.argmin(s, axis=-1)
    onehot_t = (jax.lax.broadcasted_iota(jnp.int32, (K, HW), 0)
                == idx[None, :]).astype(jnp.float32)
    quant_t = jax.lax.dot_general(
        cb, onehot_t, (((0,), (0,)), ((), ())),
        preferred_element_type=jnp.float32)
    qout_ref[0] = quant_t
    idx_ref[0, 0, :] = idx
    loss_ref[0, 0, :] = jnp.broadcast_to(
        jnp.sum(m) + jnp.sum(xb * xb), (128,))


@jax.jit
def kernel(x, codebook):
    B, Cc, H, W = x.shape
    x3 = x.reshape(B, Cc, H * W)
    qout, idx3, loss3 = pl.pallas_call(
        _vq_image_block,
        grid=(B,),
        in_specs=[
            pl.BlockSpec((1, Cc, H * W), lambda i: (i, 0, 0)),
            pl.BlockSpec((K, C), lambda i: (0, 0)),
        ],
        out_specs=[
            pl.BlockSpec((1, Cc, H * W), lambda i: (i, 0, 0)),
            pl.BlockSpec((1, 1, H * W), lambda i: (i, 0, 0)),
            pl.BlockSpec((1, 1, 128), lambda i: (i, 0, 0)),
        ],
        out_shape=[
            jax.ShapeDtypeStruct((B, Cc, H * W), jnp.float32),
            jax.ShapeDtypeStruct((B, 1, H * W), jnp.int32),
            jax.ShapeDtypeStruct((B, 1, 128), jnp.float32),
        ],
        compiler_params=pltpu.CompilerParams(
            dimension_semantics=("parallel",)),
    )(x3, codebook)
    loss = (jnp.sum(loss3[:, 0, 0]) / (B * Cc * H * W)).astype(jnp.float32)
    quant_out = qout.reshape(B, Cc, H, W)
    idx_emb = idx3.reshape(B, H * W)
    return (quant_out, loss, loss, idx_emb)


# speedup vs baseline: 1.5525x; 1.0095x over previous
import jax
import jax.numpy as jnp
from jax.experimental import pallas as pl
from jax.experimental.pallas import tpu as pltpu

K = 1024
C = 64
HW = 1024


def _vq_image_block(x_ref, cb_ref, qout_ref, idx_ref, loss_ref):
    xb = x_ref[0]
    cb = cb_ref[...]
    e_sq = jnp.sum(cb * cb, axis=-1)
    xpx = jnp.transpose(xb)
    s = jax.lax.dot_general(
        xpx, cb * -2.0, (((1,), (1,)), ((), ())),
        preferred_element_type=jnp.float32)
    s = s + e_sq[None, :]
    m = jnp.min(s, axis=-1)
    idx = jnp.argmin(s, axis=-1)
    onehot = (jax.lax.broadcasted_iota(jnp.int32, (HW, K), 1)
              == idx[:, None]).astype(jnp.float32)
    quant_t = jax.lax.dot_general(
        cb, onehot, (((0,), (1,)), ((), ())),
        preferred_element_type=jnp.float32)
    qout_ref[0] = quant_t
    idx_ref[0, 0, :] = idx
    loss_ref[0, 0, :] = jnp.broadcast_to(
        jnp.sum(m) + jnp.sum(xb * xb), (128,))


@jax.jit
def kernel(x, codebook):
    B, Cc, H, W = x.shape
    x3 = x.reshape(B, Cc, H * W)
    qout, idx3, loss3 = pl.pallas_call(
        _vq_image_block,
        grid=(B,),
        in_specs=[
            pl.BlockSpec((1, Cc, H * W), lambda i: (i, 0, 0)),
            pl.BlockSpec((K, C), lambda i: (0, 0)),
        ],
        out_specs=[
            pl.BlockSpec((1, Cc, H * W), lambda i: (i, 0, 0)),
            pl.BlockSpec((1, 1, H * W), lambda i: (i, 0, 0)),
            pl.BlockSpec((1, 1, 128), lambda i: (i, 0, 0)),
        ],
        out_shape=[
            jax.ShapeDtypeStruct((B, Cc, H * W), jnp.float32),
            jax.ShapeDtypeStruct((B, 1, H * W), jnp.int32),
            jax.ShapeDtypeStruct((B, 1, 128), jnp.float32),
        ],
        compiler_params=pltpu.CompilerParams(
            dimension_semantics=("parallel",)),
    )(x3, codebook)
    loss = (jnp.sum(loss3[:, 0, 0]) / (B * Cc * H * W)).astype(jnp.float32)
    quant_out = qout.reshape(B, Cc, H, W)
    idx_emb = idx3.reshape(B, H * W)
    return (quant_out, loss, loss, idx_emb)
